# speedup vs baseline: 5.8687x; 5.8687x over previous
import functools

import jax
import jax.numpy as jnp
from jax import lax
from jax.experimental import pallas as pl
from jax.experimental.pallas import tpu as pltpu
from jax.experimental.pallas import tpu_sc as plsc

N_NODES = 10000
NPAD = 10240
D = 256
DH = 128
E = 160000
EPAD = 163840
CHUNK = 128
NS = 16
NC = 2
ROWS_PER_TILE = NPAD // NS
DEG_CHUNKS = EPAD // (NS * NC) // CHUNK
AGG_CHUNKS = EPAD // NS // CHUNK

_mesh = plsc.VectorSubcoreMesh(core_axis_name="c", subcore_axis_name="s")


@functools.partial(
    pl.kernel,
    out_type=(
        jax.ShapeDtypeStruct((NPAD,), jnp.float32),
        jax.ShapeDtypeStruct((NPAD,), jnp.float32),
    ),
    mesh=_mesh,
    scratch_types=[
        pltpu.VMEM((DEG_CHUNKS, CHUNK), jnp.int32),
        pltpu.VMEM((CHUNK,), jnp.float32),
        pltpu.VMEM((ROWS_PER_TILE,), jnp.float32),
        pltpu.VMEM_SHARED((NPAD,), jnp.float32),
    ],
)
def _deg_call(dst_hbm, deg0_hbm, deg1_hbm, idx_v, ones_v, zeros_v, acc):
    c = lax.axis_index("c")
    s = lax.axis_index("s")
    wid = c * NS + s

    def fill(i, _):
        ones_v[pl.ds(i * 16, 16)] = jnp.full((16,), 1.0, jnp.float32)
        return 0

    lax.fori_loop(0, CHUNK // 16, fill, 0)

    def zfill(i, _):
        zeros_v[pl.ds(i * 16, 16)] = jnp.zeros((16,), jnp.float32)
        return 0

    lax.fori_loop(0, ROWS_PER_TILE // 16, zfill, 0)
    pltpu.sync_copy(zeros_v, acc.at[pl.ds(s * ROWS_PER_TILE, ROWS_PER_TILE)])
    pltpu.sync_copy(dst_hbm.at[pl.ds(wid * DEG_CHUNKS, DEG_CHUNKS)], idx_v)
    plsc.subcore_barrier()

    def chunk(j, _):
        pltpu.sync_copy(ones_v, acc.at[idx_v.at[j]], add=True)
        return 0

    lax.fori_loop(0, DEG_CHUNKS, chunk, 0)
    plsc.subcore_barrier()

    @pl.when(c == 0)
    def _():
        pltpu.sync_copy(acc.at[pl.ds(s * ROWS_PER_TILE, ROWS_PER_TILE)],
                        deg0_hbm.at[pl.ds(s * ROWS_PER_TILE, ROWS_PER_TILE)])

    @pl.when(c == 1)
    def _():
        pltpu.sync_copy(acc.at[pl.ds(s * ROWS_PER_TILE, ROWS_PER_TILE)],
                        deg1_hbm.at[pl.ds(s * ROWS_PER_TILE, ROWS_PER_TILE)])


@functools.partial(
    pl.kernel,
    out_type=(
        jax.ShapeDtypeStruct((NPAD, DH), jnp.float32),
        jax.ShapeDtypeStruct((NPAD, DH), jnp.float32),
    ),
    mesh=_mesh,
    scratch_types=[
        pltpu.VMEM((AGG_CHUNKS, CHUNK), jnp.int32),
        pltpu.VMEM((AGG_CHUNKS, CHUNK), jnp.int32),
        pltpu.VMEM((CHUNK, DH), jnp.float32),
        pltpu.VMEM_SHARED((NPAD, DH), jnp.float32),
        pltpu.SemaphoreType.DMA,
    ],
)
def _agg_call(xw0_hbm, xw1_hbm, src_hbm, dst_hbm, out0_hbm, out1_hbm,
              sidx_v, didx_v, buf_v, acc, sem):
    c = lax.axis_index("c")
    s = lax.axis_index("s")
    rows = pl.ds(s * ROWS_PER_TILE, ROWS_PER_TILE)

    @pl.when(c == 0)
    def _():
        pltpu.sync_copy(xw0_hbm.at[rows], acc.at[rows])

    @pl.when(c == 1)
    def _():
        pltpu.sync_copy(xw1_hbm.at[rows], acc.at[rows])

    pltpu.sync_copy(src_hbm.at[pl.ds(s * AGG_CHUNKS, AGG_CHUNKS)], sidx_v)
    pltpu.sync_copy(dst_hbm.at[pl.ds(s * AGG_CHUNKS, AGG_CHUNKS)], didx_v)
    plsc.subcore_barrier()

    def chunk(j, _):
        @pl.when(c == 0)
        def _():
            pltpu.async_copy(xw0_hbm.at[sidx_v.at[j]], buf_v, sem).wait()

        @pl.when(c == 1)
        def _():
            pltpu.async_copy(xw1_hbm.at[sidx_v.at[j]], buf_v, sem).wait()

        pltpu.sync_copy(buf_v, acc.at[didx_v.at[j]], add=True)
        return 0

    lax.fori_loop(0, AGG_CHUNKS, chunk, 0)
    plsc.subcore_barrier()

    @pl.when(c == 0)
    def _():
        pltpu.sync_copy(acc.at[rows], out0_hbm.at[rows])

    @pl.when(c == 1)
    def _():
        pltpu.sync_copy(acc.at[rows], out1_hbm.at[rows])


_RB = 512
_GRID = (NPAD // _RB,)


def _dinv(d0, d1):
    return lax.rsqrt(d0 + d1 + 1.0)


def _mm1_body(x_ref, w_ref, d0_ref, d1_ref, o0_ref, o1_ref):
    dinv = _dinv(d0_ref[...], d1_ref[...])
    xw = jnp.dot(x_ref[...], w_ref[...], preferred_element_type=jnp.float32)
    xw = xw * dinv[:, None]
    o0_ref[...] = xw[:, :DH]
    o1_ref[...] = xw[:, DH:]


def _mm2_body(a0_ref, a1_ref, d0_ref, d1_ref, b_ref, w_ref, o0_ref, o1_ref):
    dinv = _dinv(d0_ref[...], d1_ref[...])
    h = jnp.concatenate([a0_ref[...], a1_ref[...]], axis=1)
    h = jnp.maximum(h * dinv[:, None] + b_ref[...][None, :], 0.0)
    xw = jnp.dot(h, w_ref[...], preferred_element_type=jnp.float32)
    xw = xw * dinv[:, None]
    o0_ref[...] = xw[:, :DH]
    o1_ref[...] = xw[:, DH:]


def _final_body(a0_ref, a1_ref, d0_ref, d1_ref, b_ref, o_ref):
    dinv = _dinv(d0_ref[...], d1_ref[...])
    z = jnp.concatenate([a0_ref[...], a1_ref[...]], axis=1)
    z = z * dinv[:, None] + b_ref[...][None, :]
    m = jnp.max(z, axis=1, keepdims=True)
    lse = jnp.log(jnp.sum(jnp.exp(z - m), axis=1, keepdims=True)) + m
    o_ref[...] = z - lse


_row = pl.BlockSpec((_RB,), lambda r: (r,))
_rowh = pl.BlockSpec((_RB, DH), lambda r: (r, 0))
_rowf = pl.BlockSpec((_RB, D), lambda r: (r, 0))
_wsp = pl.BlockSpec((D, D), lambda r: (0, 0))
_bsp = pl.BlockSpec((D,), lambda r: (0,))

_mm1 = pl.pallas_call(
    _mm1_body,
    grid=_GRID,
    in_specs=[_rowf, _wsp, _row, _row],
    out_specs=[_rowh, _rowh],
    out_shape=(
        jax.ShapeDtypeStruct((NPAD, DH), jnp.float32),
        jax.ShapeDtypeStruct((NPAD, DH), jnp.float32),
    ),
)

_mm2 = pl.pallas_call(
    _mm2_body,
    grid=_GRID,
    in_specs=[_rowh, _rowh, _row, _row, _bsp, _wsp],
    out_specs=[_rowh, _rowh],
    out_shape=(
        jax.ShapeDtypeStruct((NPAD, DH), jnp.float32),
        jax.ShapeDtypeStruct((NPAD, DH), jnp.float32),
    ),
)

_final = pl.pallas_call(
    _final_body,
    grid=_GRID,
    in_specs=[_rowh, _rowh, _row, _row, _bsp],
    out_specs=_rowf,
    out_shape=jax.ShapeDtypeStruct((NPAD, D), jnp.float32),
)


def kernel(graph, nfeat, W1, b1, W2, b2, W3, b3):
    src = graph[0].astype(jnp.int32)
    dst = graph[1].astype(jnp.int32)
    srcp = jnp.concatenate(
        [src, jnp.zeros((EPAD - E,), jnp.int32)]).reshape(EPAD // CHUNK, CHUNK)
    dstp = jnp.concatenate(
        [dst, jnp.full((EPAD - E,), N_NODES, jnp.int32)]).reshape(
            EPAD // CHUNK, CHUNK)
    x = jnp.concatenate(
        [nfeat, jnp.zeros((NPAD - N_NODES, D), jnp.float32)], axis=0)

    deg0, deg1 = _deg_call(dstp)
    xw0, xw1 = _mm1(x, W1, deg0, deg1)
    a0, a1 = _agg_call(xw0, xw1, srcp, dstp)
    xw0, xw1 = _mm2(a0, a1, deg0, deg1, b1, W2)
    a0, a1 = _agg_call(xw0, xw1, srcp, dstp)
    xw0, xw1 = _mm2(a0, a1, deg0, deg1, b2, W3)
    a0, a1 = _agg_call(xw0, xw1, srcp, dstp)
    out = _final(a0, a1, deg0, deg1, b3)
    return out[:N_NODES]


# speedup vs baseline: 7.6696x; 1.3069x over previous
import functools

import jax
import jax.numpy as jnp
from jax import lax
from jax.experimental import pallas as pl
from jax.experimental.pallas import tpu as pltpu
from jax.experimental.pallas import tpu_sc as plsc

N_NODES = 10000
NPAD = 10240
D = 256
DH = 128
E = 160000
EPAD = 163840
CHUNK = 128
NS = 16
NC = 2
ROWS_PER_TILE = NPAD // NS
DEG_CHUNKS = EPAD // (NS * NC) // CHUNK
AGG_CHUNKS = EPAD // NS // CHUNK

_mesh = plsc.VectorSubcoreMesh(core_axis_name="c", subcore_axis_name="s")


@functools.partial(
    pl.kernel,
    out_type=(
        jax.ShapeDtypeStruct((NPAD,), jnp.float32),
        jax.ShapeDtypeStruct((NPAD,), jnp.float32),
    ),
    mesh=_mesh,
    scratch_types=[
        pltpu.VMEM((DEG_CHUNKS, CHUNK), jnp.int32),
        pltpu.VMEM((CHUNK,), jnp.float32),
        pltpu.VMEM((ROWS_PER_TILE,), jnp.float32),
        pltpu.VMEM_SHARED((NPAD,), jnp.float32),
    ],
)
def _deg_call(dst_hbm, deg0_hbm, deg1_hbm, idx_v, ones_v, zeros_v, acc):
    c = lax.axis_index("c")
    s = lax.axis_index("s")
    wid = c * NS + s

    def fill(i, _):
        ones_v[pl.ds(i * 16, 16)] = jnp.full((16,), 1.0, jnp.float32)
        return 0

    lax.fori_loop(0, CHUNK // 16, fill, 0)

    def zfill(i, _):
        zeros_v[pl.ds(i * 16, 16)] = jnp.zeros((16,), jnp.float32)
        return 0

    lax.fori_loop(0, ROWS_PER_TILE // 16, zfill, 0)
    pltpu.sync_copy(zeros_v, acc.at[pl.ds(s * ROWS_PER_TILE, ROWS_PER_TILE)])
    pltpu.sync_copy(dst_hbm.at[pl.ds(wid * DEG_CHUNKS, DEG_CHUNKS)], idx_v)
    plsc.subcore_barrier()

    def chunk(j, _):
        pltpu.sync_copy(ones_v, acc.at[idx_v.at[j]], add=True)
        return 0

    lax.fori_loop(0, DEG_CHUNKS, chunk, 0)
    plsc.subcore_barrier()

    @pl.when(c == 0)
    def _():
        pltpu.sync_copy(acc.at[pl.ds(s * ROWS_PER_TILE, ROWS_PER_TILE)],
                        deg0_hbm.at[pl.ds(s * ROWS_PER_TILE, ROWS_PER_TILE)])

    @pl.when(c == 1)
    def _():
        pltpu.sync_copy(acc.at[pl.ds(s * ROWS_PER_TILE, ROWS_PER_TILE)],
                        deg1_hbm.at[pl.ds(s * ROWS_PER_TILE, ROWS_PER_TILE)])


NBUF = 2
IDX_MASK = 16383
IDX_SHIFT = 14


@functools.partial(
    pl.kernel,
    out_type=(
        jax.ShapeDtypeStruct((NPAD, DH), jnp.float32),
        jax.ShapeDtypeStruct((NPAD, DH), jnp.float32),
    ),
    mesh=_mesh,
    scratch_types=[
        pltpu.VMEM((AGG_CHUNKS, CHUNK), jnp.int32),
        [pltpu.VMEM((CHUNK,), jnp.int32)] * NBUF,
        [pltpu.VMEM((CHUNK,), jnp.int32)] * NBUF,
        [pltpu.VMEM((CHUNK, DH), jnp.float32)] * NBUF,
        [pltpu.SemaphoreType.DMA] * NBUF,
        pltpu.VMEM_SHARED((NPAD, DH), jnp.float32),
    ],
)
def _agg_call(xw0_hbm, xw1_hbm, pk_hbm, out0_hbm, out1_hbm,
              pk_v, us, ud, bufs, sems, acc):
    c = lax.axis_index("c")
    s = lax.axis_index("s")
    rows = pl.ds(s * ROWS_PER_TILE, ROWS_PER_TILE)

    @pl.when(c == 0)
    def _():
        pltpu.sync_copy(xw0_hbm.at[rows], acc.at[rows])

    @pl.when(c == 1)
    def _():
        pltpu.sync_copy(xw1_hbm.at[rows], acc.at[rows])

    pltpu.sync_copy(pk_hbm.at[pl.ds(s * AGG_CHUNKS, AGG_CHUNKS)], pk_v)
    plsc.subcore_barrier()

    def unpack(j, b):
        row = pk_v.at[j]

        def u(i, _):
            sl = pl.ds(i * 16, 16)
            v = row[sl]
            us[b][sl] = v & IDX_MASK
            ud[b][sl] = lax.shift_right_logical(v, IDX_SHIFT)
            return 0

        lax.fori_loop(0, CHUNK // 16, u, 0)

    def gather_start(b):
        @pl.when(c == 0)
        def _():
            pltpu.make_async_copy(
                xw0_hbm.at[us[b]], bufs[b], sems[b]).start()

        @pl.when(c == 1)
        def _():
            pltpu.make_async_copy(
                xw1_hbm.at[us[b]], bufs[b], sems[b]).start()

    def gather_wait(b):
        @pl.when(c == 0)
        def _():
            pltpu.make_async_copy(
                xw0_hbm.at[us[b]], bufs[b], sems[b]).wait()

        @pl.when(c == 1)
        def _():
            pltpu.make_async_copy(
                xw1_hbm.at[us[b]], bufs[b], sems[b]).wait()

    for b in range(NBUF):
        unpack(b, b)
        gather_start(b)

    def body(jj, _):
        for b in range(NBUF):
            j = jj * NBUF + b
            gather_wait(b)
            pltpu.sync_copy(bufs[b], acc.at[ud[b]], add=True)

            @pl.when(j + NBUF < AGG_CHUNKS)
            def _():
                unpack(j + NBUF, b)
                gather_start(b)

        return 0

    lax.fori_loop(0, AGG_CHUNKS // NBUF, body, 0)
    plsc.subcore_barrier()

    @pl.when(c == 0)
    def _():
        pltpu.sync_copy(acc.at[rows], out0_hbm.at[rows])

    @pl.when(c == 1)
    def _():
        pltpu.sync_copy(acc.at[rows], out1_hbm.at[rows])


_RB = 512
_GRID = (NPAD // _RB,)


def _dinv(d0, d1):
    return lax.rsqrt(d0 + d1 + 1.0)


def _mm1_body(x_ref, w_ref, d0_ref, d1_ref, o0_ref, o1_ref):
    dinv = _dinv(d0_ref[...], d1_ref[...])
    xw = jnp.dot(x_ref[...], w_ref[...], preferred_element_type=jnp.float32)
    xw = xw * dinv[:, None]
    o0_ref[...] = xw[:, :DH]
    o1_ref[...] = xw[:, DH:]


def _mm2_body(a0_ref, a1_ref, d0_ref, d1_ref, b_ref, w_ref, o0_ref, o1_ref):
    dinv = _dinv(d0_ref[...], d1_ref[...])
    h = jnp.concatenate([a0_ref[...], a1_ref[...]], axis=1)
    h = jnp.maximum(h * dinv[:, None] + b_ref[...][None, :], 0.0)
    xw = jnp.dot(h, w_ref[...], preferred_element_type=jnp.float32)
    xw = xw * dinv[:, None]
    o0_ref[...] = xw[:, :DH]
    o1_ref[...] = xw[:, DH:]


def _final_body(a0_ref, a1_ref, d0_ref, d1_ref, b_ref, o_ref):
    dinv = _dinv(d0_ref[...], d1_ref[...])
    z = jnp.concatenate([a0_ref[...], a1_ref[...]], axis=1)
    z = z * dinv[:, None] + b_ref[...][None, :]
    m = jnp.max(z, axis=1, keepdims=True)
    lse = jnp.log(jnp.sum(jnp.exp(z - m), axis=1, keepdims=True)) + m
    o_ref[...] = z - lse


_row = pl.BlockSpec((_RB,), lambda r: (r,))
_rowh = pl.BlockSpec((_RB, DH), lambda r: (r, 0))
_rowf = pl.BlockSpec((_RB, D), lambda r: (r, 0))
_wsp = pl.BlockSpec((D, D), lambda r: (0, 0))
_bsp = pl.BlockSpec((D,), lambda r: (0,))

_mm1 = pl.pallas_call(
    _mm1_body,
    grid=_GRID,
    in_specs=[_rowf, _wsp, _row, _row],
    out_specs=[_rowh, _rowh],
    out_shape=(
        jax.ShapeDtypeStruct((NPAD, DH), jnp.float32),
        jax.ShapeDtypeStruct((NPAD, DH), jnp.float32),
    ),
)

_mm2 = pl.pallas_call(
    _mm2_body,
    grid=_GRID,
    in_specs=[_rowh, _rowh, _row, _row, _bsp, _wsp],
    out_specs=[_rowh, _rowh],
    out_shape=(
        jax.ShapeDtypeStruct((NPAD, DH), jnp.float32),
        jax.ShapeDtypeStruct((NPAD, DH), jnp.float32),
    ),
)

_final = pl.pallas_call(
    _final_body,
    grid=_GRID,
    in_specs=[_rowh, _rowh, _row, _row, _bsp],
    out_specs=_rowf,
    out_shape=jax.ShapeDtypeStruct((NPAD, D), jnp.float32),
)


def kernel(graph, nfeat, W1, b1, W2, b2, W3, b3):
    src = graph[0].astype(jnp.int32)
    dst = graph[1].astype(jnp.int32)
    srcp = jnp.concatenate(
        [src, jnp.zeros((EPAD - E,), jnp.int32)]).reshape(EPAD // CHUNK, CHUNK)
    dstp = jnp.concatenate(
        [dst, jnp.full((EPAD - E,), N_NODES, jnp.int32)]).reshape(
            EPAD // CHUNK, CHUNK)
    packed = dstp * (IDX_MASK + 1) + srcp
    x = jnp.concatenate(
        [nfeat, jnp.zeros((NPAD - N_NODES, D), jnp.float32)], axis=0)

    deg0, deg1 = _deg_call(dstp)
    xw0, xw1 = _mm1(x, W1, deg0, deg1)
    a0, a1 = _agg_call(xw0, xw1, packed)
    xw0, xw1 = _mm2(a0, a1, deg0, deg1, b1, W2)
    a0, a1 = _agg_call(xw0, xw1, packed)
    xw0, xw1 = _mm2(a0, a1, deg0, deg1, b2, W3)
    a0, a1 = _agg_call(xw0, xw1, packed)
    out = _final(a0, a1, deg0, deg1, b3)
    return out[:N_NODES]
